# Initial kernel scaffold; baseline (speedup 1.0000x reference)
import jax
import jax.numpy as jnp
from jax.experimental import pallas as pl


def kernel(pos, z, x, edge_index, batch_index, subg_node_index, subg_node_center_index, subg_edge_index, subg_batch_index, subg_node_label, emb_table, W_ef, W_msg, W_upd, b_upd, W_out):
    raise NotImplementedError("write your pallas kernel here")


# speedup vs baseline: 1.6393x; 1.6393x over previous
import functools

import jax
import jax.numpy as jnp
from jax import lax
from jax.experimental import pallas as pl
from jax.experimental.pallas import tpu as pltpu
from jax.experimental.pallas import tpu_sc as plsc

N = 10000
E = 320000
NG = 64
HD = 128
EF = 16
MAXZ = 100
CUT = 10.0
RBOUND = 10.0
LAYERS = 4
C = 1.0
Y_STD = 1.0
Y_MEAN = 0.0

NC = 2
NS = 16
NW = NC * NS
EPT = E // NW
K = 80
NCH = EPT // K
SPT = N // NS

_mesh = plsc.VectorSubcoreMesh(core_axis_name="c", subcore_axis_name="s")
_sc_params = pltpu.CompilerParams(use_tc_tiling_on_sc=False)


def _silu(x):
    return x * (1.0 / (1.0 + jnp.exp(-x)))


@functools.partial(
    pl.kernel,
    out_type=[
        jax.ShapeDtypeStruct((E, 16), jnp.float32),
        jax.ShapeDtypeStruct((E, 16), jnp.float32),
    ],
    mesh=_mesh,
    scratch_types=[
        pltpu.VMEM((NCH, K), jnp.int32),
        pltpu.VMEM((NCH, K), jnp.int32),
        pltpu.VMEM((K, 16), jnp.float32),
        pltpu.VMEM((K, 16), jnp.float32),
        pltpu.SemaphoreType.DMA,
        pltpu.SemaphoreType.DMA,
    ],
    compiler_params=_sc_params,
)
def _sc_geom(pos_hbm, src_hbm, dst_hbm, ps_hbm, pd_hbm,
             src_v, dst_v, ps_v, pd_v, sem_a, sem_b):
    cid = lax.axis_index("c")
    sid = lax.axis_index("s")
    wid = cid * NS + sid
    base = wid * EPT
    pltpu.sync_copy(src_hbm.at[wid], src_v)
    pltpu.sync_copy(dst_hbm.at[wid], dst_v)

    @pl.loop(0, NCH)
    def _chunk(j):
        ca = pltpu.async_copy(pos_hbm.at[src_v.at[j]], ps_v, sem_a)
        cb = pltpu.async_copy(pos_hbm.at[dst_v.at[j]], pd_v, sem_b)
        ca.wait()
        cb.wait()
        pltpu.sync_copy(ps_v, ps_hbm.at[pl.ds(base + j * K, K)])
        pltpu.sync_copy(pd_v, pd_hbm.at[pl.ds(base + j * K, K)])


@functools.partial(
    pl.kernel,
    out_type=jax.ShapeDtypeStruct((NC, N, HD), jnp.float32),
    mesh=_mesh,
    scratch_types=[
        pltpu.VMEM((NCH, K), jnp.int32),
        pltpu.VMEM((NCH, K), jnp.int32),
        pltpu.VMEM((K, HD), jnp.float32),
        pltpu.VMEM((K, HD), jnp.float32),
        pltpu.VMEM_SHARED((N, HD), jnp.float32),
        pltpu.SemaphoreType.DMA,
    ],
    compiler_params=_sc_params,
)
def _sc_msg(msg_hbm, gate_hbm, src_hbm, dst_hbm, zeros_hbm, agg2_hbm,
            src_v, dst_v, rows_v, gate_v, agg_sh, gsem):
    cid = lax.axis_index("c")
    sid = lax.axis_index("s")
    wid = cid * NS + sid
    base = wid * EPT
    pltpu.sync_copy(zeros_hbm.at[pl.ds(sid * SPT, SPT)],
                    agg_sh.at[pl.ds(sid * SPT, SPT)])
    pltpu.sync_copy(src_hbm.at[wid], src_v)
    pltpu.sync_copy(dst_hbm.at[wid], dst_v)
    plsc.subcore_barrier()

    @pl.loop(0, NCH)
    def _chunk(j):
        pltpu.sync_copy(gate_hbm.at[pl.ds(base + j * K, K)], gate_v)
        pltpu.async_copy(msg_hbm.at[src_v.at[j]], rows_v, gsem).wait()

        @pl.loop(0, K)
        def _row(i):
            for k in range(HD // 16):
                sl = pl.ds(k * 16, 16)
                rows_v[i, sl] = rows_v[i, sl] * gate_v[i, sl]

        pltpu.sync_copy(rows_v, agg_sh.at[dst_v.at[j]], add=True)

    plsc.subcore_barrier()
    pltpu.sync_copy(agg_sh.at[pl.ds(sid * SPT, SPT)],
                    agg2_hbm.at[cid, pl.ds(sid * SPT, SPT)])


EB = 2000


def _tc_gates_body(ps_ref, pd_ref, wef_ref, out_ref):
    diff = ps_ref[...] - pd_ref[...]
    d2 = jnp.sum(diff * diff, axis=1)
    dist = jnp.sqrt(d2 + 1e-12)
    step = RBOUND / (EF - 1)
    centers = lax.broadcasted_iota(jnp.int32, (EB, EF), 1).astype(jnp.float32) * step
    gamma = (EF / RBOUND) ** 2
    ef = jnp.exp(-gamma * (dist[:, None] - centers) ** 2)
    cut = 0.5 * (jnp.cos(jnp.pi * jnp.clip(dist / CUT, 0.0, 1.0)) + 1.0)
    cut = cut * (dist < CUT).astype(jnp.float32)
    g = _silu(jnp.dot(ef, wef_ref[0], preferred_element_type=jnp.float32))
    out_ref[0] = g * cut[:, None]


def _tc_gates(ps, pd, w_ef):
    return pl.pallas_call(
        _tc_gates_body,
        grid=(LAYERS, E // EB),
        in_specs=[
            pl.BlockSpec((EB, 16), lambda l, j: (j, 0)),
            pl.BlockSpec((EB, 16), lambda l, j: (j, 0)),
            pl.BlockSpec((1, EF, HD), lambda l, j: (l, 0, 0)),
        ],
        out_specs=pl.BlockSpec((1, EB, HD), lambda l, j: (l, j, 0)),
        out_shape=jax.ShapeDtypeStruct((LAYERS, E, HD), jnp.float32),
    )(ps, pd, w_ef)


NB = 1000


def _tc_pre_body(z_ref, emb_ref, wm_ref, s_ref, m_ref):
    zb = z_ref[...]
    oh = (zb == lax.broadcasted_iota(jnp.int32, (NB, HD), 1)).astype(jnp.float32)
    s = jnp.dot(oh, emb_ref[...], preferred_element_type=jnp.float32)
    s_ref[...] = s
    m_ref[...] = jnp.dot(s, wm_ref[...], preferred_element_type=jnp.float32)


def _tc_pre(z2, emb_pad, wm0):
    return pl.pallas_call(
        _tc_pre_body,
        grid=(N // NB,),
        in_specs=[
            pl.BlockSpec((NB, 1), lambda j: (j, 0)),
            pl.BlockSpec((HD, HD), lambda j: (0, 0)),
            pl.BlockSpec((HD, HD), lambda j: (0, 0)),
        ],
        out_specs=[
            pl.BlockSpec((NB, HD), lambda j: (j, 0)),
            pl.BlockSpec((NB, HD), lambda j: (j, 0)),
        ],
        out_shape=[
            jax.ShapeDtypeStruct((N, HD), jnp.float32),
            jax.ShapeDtypeStruct((N, HD), jnp.float32),
        ],
    )(z2, emb_pad, wm0)


def _tc_upd_body(s_ref, a0_ref, a1_ref, wus_ref, wua_ref, b_ref, wm_ref,
                 s2_ref, m_ref):
    agg = a0_ref[0] + a1_ref[0]
    pre = (jnp.dot(s_ref[...], wus_ref[...], preferred_element_type=jnp.float32)
           + jnp.dot(agg, wua_ref[...], preferred_element_type=jnp.float32)
           + b_ref[...])
    s2 = s_ref[...] + _silu(pre)
    s2_ref[...] = s2
    m_ref[...] = jnp.dot(s2, wm_ref[...], preferred_element_type=jnp.float32)


def _tc_upd(scalar, agg2, wus, wua, b, wm_next):
    return pl.pallas_call(
        _tc_upd_body,
        grid=(N // NB,),
        in_specs=[
            pl.BlockSpec((NB, HD), lambda j: (j, 0)),
            pl.BlockSpec((1, NB, HD), lambda j: (0, j, 0)),
            pl.BlockSpec((1, NB, HD), lambda j: (1, j, 0)),
            pl.BlockSpec((HD, HD), lambda j: (0, 0)),
            pl.BlockSpec((HD, HD), lambda j: (0, 0)),
            pl.BlockSpec((1, HD), lambda j: (0, 0)),
            pl.BlockSpec((HD, HD), lambda j: (0, 0)),
        ],
        out_specs=[
            pl.BlockSpec((NB, HD), lambda j: (j, 0)),
            pl.BlockSpec((NB, HD), lambda j: (j, 0)),
        ],
        out_shape=[
            jax.ShapeDtypeStruct((N, HD), jnp.float32),
            jax.ShapeDtypeStruct((N, HD), jnp.float32),
        ],
    )(scalar, agg2, agg2, wus, wua, b, wm_next)


def _tc_final_body(s_ref, a0_ref, a1_ref, wus_ref, wua_ref, b_ref, bi_ref,
                   wo_ref, acc_ref, pred_ref):
    j = pl.program_id(0)

    @pl.when(j == 0)
    def _():
        acc_ref[...] = jnp.zeros((NG, HD), jnp.float32)

    agg = a0_ref[0] + a1_ref[0]
    pre = (jnp.dot(s_ref[...], wus_ref[...], preferred_element_type=jnp.float32)
           + jnp.dot(agg, wua_ref[...], preferred_element_type=jnp.float32)
           + b_ref[...])
    s2 = s_ref[...] + _silu(pre)
    oh = (bi_ref[...] == lax.broadcasted_iota(jnp.int32, (NB, NG), 1)).astype(jnp.float32)
    acc_ref[...] += lax.dot_general(oh, s2, (((0,), (0,)), ((), ())),
                                    preferred_element_type=jnp.float32)

    @pl.when(j == pl.num_programs(0) - 1)
    def _():
        graph = acc_ref[...] * C
        pred = jnp.sum(graph * wo_ref[...], axis=1, keepdims=True)
        pred_ref[...] = pred * Y_STD + Y_MEAN


def _tc_final(scalar, agg2, wus, wua, b, bi2, wo_t):
    return pl.pallas_call(
        _tc_final_body,
        grid=(N // NB,),
        in_specs=[
            pl.BlockSpec((NB, HD), lambda j: (j, 0)),
            pl.BlockSpec((1, NB, HD), lambda j: (0, j, 0)),
            pl.BlockSpec((1, NB, HD), lambda j: (1, j, 0)),
            pl.BlockSpec((HD, HD), lambda j: (0, 0)),
            pl.BlockSpec((HD, HD), lambda j: (0, 0)),
            pl.BlockSpec((1, HD), lambda j: (0, 0)),
            pl.BlockSpec((NB, 1), lambda j: (j, 0)),
            pl.BlockSpec((1, HD), lambda j: (0, 0)),
        ],
        out_specs=[
            pl.BlockSpec((NG, HD), lambda j: (0, 0)),
            pl.BlockSpec((NG, 1), lambda j: (0, 0)),
        ],
        out_shape=[
            jax.ShapeDtypeStruct((NG, HD), jnp.float32),
            jax.ShapeDtypeStruct((NG, 1), jnp.float32),
        ],
    )(scalar, agg2, agg2, wus, wua, b, bi2, wo_t)


def kernel(pos, z, x, edge_index, batch_index, subg_node_index,
           subg_node_center_index, subg_edge_index, subg_batch_index,
           subg_node_label, emb_table, W_ef, W_msg, W_upd, b_upd, W_out):
    src = edge_index[0].astype(jnp.int32).reshape(NW, NCH, K)
    dst = edge_index[1].astype(jnp.int32).reshape(NW, NCH, K)
    zeros_nh = jnp.zeros((N, HD), jnp.float32)
    emb_pad = jnp.zeros((HD, HD), jnp.float32).at[:MAXZ].set(emb_table)

    pos16 = jnp.pad(pos.astype(jnp.float32), ((0, 0), (0, 13)))
    ps, pd = _sc_geom(pos16, src, dst)
    gates = _tc_gates(ps, pd, W_ef)

    scalar, node_msg = _tc_pre(z.astype(jnp.int32).reshape(N, 1), emb_pad, W_msg[0])

    for l in range(LAYERS):
        agg2 = _sc_msg(node_msg, gates[l], src, dst, zeros_nh)
        wus = W_upd[l, :HD]
        wua = W_upd[l, HD:]
        b = b_upd[l].reshape(1, HD)
        if l < LAYERS - 1:
            scalar, node_msg = _tc_upd(scalar, agg2, wus, wua, b, W_msg[l + 1])
        else:
            _, pred = _tc_final(scalar, agg2, wus, wua, b,
                                batch_index.astype(jnp.int32).reshape(N, 1),
                                W_out.reshape(1, HD))
    return pred


# speedup vs baseline: 2.5191x; 1.5367x over previous
import functools

import jax
import jax.numpy as jnp
from jax import lax
from jax.experimental import pallas as pl
from jax.experimental.pallas import tpu as pltpu
from jax.experimental.pallas import tpu_sc as plsc

N = 10000
E = 320000
NG = 64
HD = 128
EF = 16
MAXZ = 100
CUT = 10.0
RBOUND = 10.0
LAYERS = 4
C = 1.0
Y_STD = 1.0
Y_MEAN = 0.0

NC = 2
NS = 16
NW = NC * NS
EPT = E // NW
K = 80
NCH = EPT // K
K2 = 40
NCH2 = EPT // K2
SPT = N // NS

_mesh = plsc.VectorSubcoreMesh(core_axis_name="c", subcore_axis_name="s")
_sc_params = pltpu.CompilerParams(use_tc_tiling_on_sc=False)


def _silu(x):
    return x * (1.0 / (1.0 + jnp.exp(-x)))


@functools.partial(
    pl.kernel,
    out_type=[
        jax.ShapeDtypeStruct((E, 16), jnp.float32),
        jax.ShapeDtypeStruct((E, 16), jnp.float32),
    ],
    mesh=_mesh,
    scratch_types=[
        pltpu.VMEM((NCH, K), jnp.int32),
        pltpu.VMEM((NCH, K), jnp.int32),
        pltpu.VMEM((K, 16), jnp.float32),
        pltpu.VMEM((K, 16), jnp.float32),
        pltpu.SemaphoreType.DMA,
        pltpu.SemaphoreType.DMA,
    ],
    compiler_params=_sc_params,
)
def _sc_geom(pos_hbm, src_hbm, dst_hbm, ps_hbm, pd_hbm,
             src_v, dst_v, ps_v, pd_v, sem_a, sem_b):
    cid = lax.axis_index("c")
    sid = lax.axis_index("s")
    wid = cid * NS + sid
    base = wid * EPT
    pltpu.sync_copy(src_hbm.at[wid], src_v)
    pltpu.sync_copy(dst_hbm.at[wid], dst_v)

    @pl.loop(0, NCH)
    def _chunk(j):
        ca = pltpu.async_copy(pos_hbm.at[src_v.at[j]], ps_v, sem_a)
        cb = pltpu.async_copy(pos_hbm.at[dst_v.at[j]], pd_v, sem_b)
        ca.wait()
        cb.wait()
        pltpu.sync_copy(ps_v, ps_hbm.at[pl.ds(base + j * K, K)])
        pltpu.sync_copy(pd_v, pd_hbm.at[pl.ds(base + j * K, K)])


@functools.partial(
    pl.kernel,
    out_type=jax.ShapeDtypeStruct((NC, N, HD), jnp.float32),
    mesh=_mesh,
    scratch_types=[
        pltpu.VMEM((NCH2, K2), jnp.int32),
        pltpu.VMEM((NCH2, K2), jnp.int32),
        pltpu.VMEM((K2, HD), jnp.float32),
        pltpu.VMEM((K2, HD), jnp.float32),
        pltpu.VMEM((K2, HD), jnp.float32),
        pltpu.VMEM((K2, HD), jnp.float32),
        pltpu.VMEM_SHARED((N, HD), jnp.float32),
        pltpu.SemaphoreType.DMA,
        pltpu.SemaphoreType.DMA,
        pltpu.SemaphoreType.DMA,
        pltpu.SemaphoreType.DMA,
    ],
    compiler_params=_sc_params,
)
def _sc_msg(msg_hbm, gate_hbm, src_hbm, dst_hbm, zeros_hbm, agg2_hbm,
            src_v, dst_v, rows_v0, rows_v1, gate_v0, gate_v1, agg_sh,
            sem_r0, sem_r1, sem_g0, sem_g1):
    rows = (rows_v0, rows_v1)
    gbuf = (gate_v0, gate_v1)
    sem_r = (sem_r0, sem_r1)
    sem_g = (sem_g0, sem_g1)
    cid = lax.axis_index("c")
    sid = lax.axis_index("s")
    wid = cid * NS + sid
    base = wid * EPT
    pltpu.sync_copy(zeros_hbm.at[pl.ds(sid * SPT, SPT)],
                    agg_sh.at[pl.ds(sid * SPT, SPT)])
    pltpu.sync_copy(src_hbm.at[wid], src_v)
    pltpu.sync_copy(dst_hbm.at[wid], dst_v)
    plsc.subcore_barrier()

    def issue(j, b):
        pltpu.async_copy(gate_hbm.at[pl.ds(base + j * K2, K2)], gbuf[b], sem_g[b])
        pltpu.async_copy(msg_hbm.at[src_v.at[j]], rows[b], sem_r[b])

    def work(j, b):
        pltpu.make_async_copy(gate_hbm.at[pl.ds(base + j * K2, K2)],
                              gbuf[b], sem_g[b]).wait()
        pltpu.make_async_copy(msg_hbm.at[src_v.at[j]], rows[b], sem_r[b]).wait()

        @pl.loop(0, K2)
        def _row(i):
            for k in range(HD // 16):
                sl = pl.ds(k * 16, 16)
                rows[b][i, sl] = rows[b][i, sl] * gbuf[b][i, sl]

        pltpu.sync_copy(rows[b], agg_sh.at[dst_v.at[j]], add=True)

    issue(0, 0)
    issue(1, 1)

    @pl.loop(0, (NCH2 - 2) // 2)
    def _pair(p):
        j0 = p * 2
        work(j0, 0)
        issue(j0 + 2, 0)
        work(j0 + 1, 1)
        issue(j0 + 3, 1)

    work(NCH2 - 2, 0)
    work(NCH2 - 1, 1)

    plsc.subcore_barrier()
    pltpu.sync_copy(agg_sh.at[pl.ds(sid * SPT, SPT)],
                    agg2_hbm.at[cid, pl.ds(sid * SPT, SPT)])


EB = 8000


def _tc_gates_body(ps_ref, pd_ref, wef_ref, out_ref):
    diff = ps_ref[...] - pd_ref[...]
    d2 = jnp.sum(diff * diff, axis=1)
    dist = jnp.sqrt(d2 + 1e-12)
    step = RBOUND / (EF - 1)
    centers = lax.broadcasted_iota(jnp.int32, (EB, EF), 1).astype(jnp.float32) * step
    gamma = (EF / RBOUND) ** 2
    ef = jnp.exp(-gamma * (dist[:, None] - centers) ** 2)
    cut = 0.5 * (jnp.cos(jnp.pi * jnp.clip(dist / CUT, 0.0, 1.0)) + 1.0)
    cut = cut * (dist < CUT).astype(jnp.float32)
    g = _silu(jnp.dot(ef, wef_ref[...], preferred_element_type=jnp.float32))
    out_ref[...] = g * cut[:, None]


def _tc_gates(ps, pd, w_ef_l):
    return pl.pallas_call(
        _tc_gates_body,
        grid=(E // EB,),
        in_specs=[
            pl.BlockSpec((EB, 16), lambda j: (j, 0)),
            pl.BlockSpec((EB, 16), lambda j: (j, 0)),
            pl.BlockSpec((EF, HD), lambda j: (0, 0)),
        ],
        out_specs=pl.BlockSpec((EB, HD), lambda j: (j, 0)),
        out_shape=jax.ShapeDtypeStruct((E, HD), jnp.float32),
    )(ps, pd, w_ef_l)


NB = 1000


def _tc_pre_body(z_ref, emb_ref, wm_ref, s_ref, m_ref):
    zb = z_ref[...]
    oh = (zb == lax.broadcasted_iota(jnp.int32, (NB, HD), 1)).astype(jnp.float32)
    s = jnp.dot(oh, emb_ref[...], preferred_element_type=jnp.float32)
    s_ref[...] = s
    m_ref[...] = jnp.dot(s, wm_ref[...], preferred_element_type=jnp.float32)


def _tc_pre(z2, emb_pad, wm0):
    return pl.pallas_call(
        _tc_pre_body,
        grid=(N // NB,),
        in_specs=[
            pl.BlockSpec((NB, 1), lambda j: (j, 0)),
            pl.BlockSpec((HD, HD), lambda j: (0, 0)),
            pl.BlockSpec((HD, HD), lambda j: (0, 0)),
        ],
        out_specs=[
            pl.BlockSpec((NB, HD), lambda j: (j, 0)),
            pl.BlockSpec((NB, HD), lambda j: (j, 0)),
        ],
        out_shape=[
            jax.ShapeDtypeStruct((N, HD), jnp.float32),
            jax.ShapeDtypeStruct((N, HD), jnp.float32),
        ],
    )(z2, emb_pad, wm0)


def _tc_upd_body(s_ref, a0_ref, a1_ref, wus_ref, wua_ref, b_ref, wm_ref,
                 s2_ref, m_ref):
    agg = a0_ref[0] + a1_ref[0]
    pre = (jnp.dot(s_ref[...], wus_ref[...], preferred_element_type=jnp.float32)
           + jnp.dot(agg, wua_ref[...], preferred_element_type=jnp.float32)
           + b_ref[...])
    s2 = s_ref[...] + _silu(pre)
    s2_ref[...] = s2
    m_ref[...] = jnp.dot(s2, wm_ref[...], preferred_element_type=jnp.float32)


def _tc_upd(scalar, agg2, wus, wua, b, wm_next):
    return pl.pallas_call(
        _tc_upd_body,
        grid=(N // NB,),
        in_specs=[
            pl.BlockSpec((NB, HD), lambda j: (j, 0)),
            pl.BlockSpec((1, NB, HD), lambda j: (0, j, 0)),
            pl.BlockSpec((1, NB, HD), lambda j: (1, j, 0)),
            pl.BlockSpec((HD, HD), lambda j: (0, 0)),
            pl.BlockSpec((HD, HD), lambda j: (0, 0)),
            pl.BlockSpec((1, HD), lambda j: (0, 0)),
            pl.BlockSpec((HD, HD), lambda j: (0, 0)),
        ],
        out_specs=[
            pl.BlockSpec((NB, HD), lambda j: (j, 0)),
            pl.BlockSpec((NB, HD), lambda j: (j, 0)),
        ],
        out_shape=[
            jax.ShapeDtypeStruct((N, HD), jnp.float32),
            jax.ShapeDtypeStruct((N, HD), jnp.float32),
        ],
    )(scalar, agg2, agg2, wus, wua, b, wm_next)


def _tc_final_body(s_ref, a0_ref, a1_ref, wus_ref, wua_ref, b_ref, bi_ref,
                   wo_ref, acc_ref, pred_ref):
    j = pl.program_id(0)

    @pl.when(j == 0)
    def _():
        acc_ref[...] = jnp.zeros((NG, HD), jnp.float32)

    agg = a0_ref[0] + a1_ref[0]
    pre = (jnp.dot(s_ref[...], wus_ref[...], preferred_element_type=jnp.float32)
           + jnp.dot(agg, wua_ref[...], preferred_element_type=jnp.float32)
           + b_ref[...])
    s2 = s_ref[...] + _silu(pre)
    oh = (bi_ref[...] == lax.broadcasted_iota(jnp.int32, (NB, NG), 1)).astype(jnp.float32)
    acc_ref[...] += lax.dot_general(oh, s2, (((0,), (0,)), ((), ())),
                                    preferred_element_type=jnp.float32)

    @pl.when(j == pl.num_programs(0) - 1)
    def _():
        graph = acc_ref[...] * C
        pred = jnp.sum(graph * wo_ref[...], axis=1, keepdims=True)
        pred_ref[...] = pred * Y_STD + Y_MEAN


def _tc_final(scalar, agg2, wus, wua, b, bi2, wo_t):
    return pl.pallas_call(
        _tc_final_body,
        grid=(N // NB,),
        in_specs=[
            pl.BlockSpec((NB, HD), lambda j: (j, 0)),
            pl.BlockSpec((1, NB, HD), lambda j: (0, j, 0)),
            pl.BlockSpec((1, NB, HD), lambda j: (1, j, 0)),
            pl.BlockSpec((HD, HD), lambda j: (0, 0)),
            pl.BlockSpec((HD, HD), lambda j: (0, 0)),
            pl.BlockSpec((1, HD), lambda j: (0, 0)),
            pl.BlockSpec((NB, 1), lambda j: (j, 0)),
            pl.BlockSpec((1, HD), lambda j: (0, 0)),
        ],
        out_specs=[
            pl.BlockSpec((NG, HD), lambda j: (0, 0)),
            pl.BlockSpec((NG, 1), lambda j: (0, 0)),
        ],
        out_shape=[
            jax.ShapeDtypeStruct((NG, HD), jnp.float32),
            jax.ShapeDtypeStruct((NG, 1), jnp.float32),
        ],
    )(scalar, agg2, agg2, wus, wua, b, bi2, wo_t)


def kernel(pos, z, x, edge_index, batch_index, subg_node_index,
           subg_node_center_index, subg_edge_index, subg_batch_index,
           subg_node_label, emb_table, W_ef, W_msg, W_upd, b_upd, W_out):
    src = edge_index[0].astype(jnp.int32).reshape(NW, NCH, K)
    dst = edge_index[1].astype(jnp.int32).reshape(NW, NCH, K)
    src2 = edge_index[0].astype(jnp.int32).reshape(NW, NCH2, K2)
    dst2 = edge_index[1].astype(jnp.int32).reshape(NW, NCH2, K2)
    zeros_nh = jnp.zeros((N, HD), jnp.float32)
    emb_pad = jnp.zeros((HD, HD), jnp.float32).at[:MAXZ].set(emb_table)

    pos16 = jnp.pad(pos.astype(jnp.float32), ((0, 0), (0, 13)))
    ps, pd = _sc_geom(pos16, src, dst)
    gates = [_tc_gates(ps, pd, W_ef[l]) for l in range(LAYERS)]

    scalar, node_msg = _tc_pre(z.astype(jnp.int32).reshape(N, 1), emb_pad, W_msg[0])

    for l in range(LAYERS):
        agg2 = _sc_msg(node_msg, gates[l], src2, dst2, zeros_nh)
        wus = W_upd[l, :HD]
        wua = W_upd[l, HD:]
        b = b_upd[l].reshape(1, HD)
        if l < LAYERS - 1:
            scalar, node_msg = _tc_upd(scalar, agg2, wus, wua, b, W_msg[l + 1])
        else:
            _, pred = _tc_final(scalar, agg2, wus, wua, b,
                                batch_index.astype(jnp.int32).reshape(N, 1),
                                W_out.reshape(1, HD))
    return pred
